# Initial kernel scaffold; baseline (speedup 1.0000x reference)
import jax
import jax.numpy as jnp
from jax.experimental import pallas as pl


def kernel(pred_boxes, fts):
    raise NotImplementedError("write your pallas kernel here")


# speedup vs baseline: 2.5148x; 2.5148x over previous
import jax
import jax.numpy as jnp
from jax import lax
from jax.experimental import pallas as pl

_CONF = 0.4
_IOU = 0.5
_MAXD = 100
_K = 4096
_IMG = 640.0
_MAXWH = 4096.0
_LANES = 128
_R = 200
_NPAD = _R * _LANES


def _nms_body(p_ref, o_ref):
    p = p_ref[...]
    obj = p[4]
    ca = obj[None, :, :] * p[5:85]
    m = jnp.max(ca, axis=0)
    ci = lax.broadcasted_iota(jnp.int32, ca.shape, 0).astype(jnp.float32)
    cid = jnp.min(jnp.where(ca == m[None], ci, jnp.float32(1e9)), axis=0)
    conf = jnp.where(m > _CONF, m, jnp.float32(0.0))

    xs = p[0] * _IMG
    ys = p[1] * _IMG
    ws = p[2] * _IMG
    hs = p[3] * _IMG
    x1 = xs - ws / 2
    y1 = ys - hs / 2
    x2 = xs + ws / 2
    y2 = ys + hs / 2
    off = cid * _MAXWH
    x1o = x1 + off
    y1o = y1 + off
    x2o = x2 + off
    y2o = y2 + off
    a2 = (x2o - x1o) * (y2o - y1o)

    nidx = (lax.broadcasted_iota(jnp.int32, (_R, _LANES), 0) * _LANES
            + lax.broadcasted_iota(jnp.int32, (_R, _LANES), 1))
    bits = lax.bitcast_convert_type(conf, jnp.int32)

    def _tau_step(t, acc):
        cand = acc | lax.shift_left(jnp.int32(1), jnp.int32(30) - t)
        cnt = jnp.sum((bits >= cand).astype(jnp.int32))
        return jnp.where(cnt >= _K, cand, acc)

    tau = lax.fori_loop(0, 31, _tau_step, jnp.int32(0))
    count_gt = jnp.sum((bits > tau).astype(jnp.int32))
    need = _K - count_gt
    eq = bits == tau

    def _idx_step(t, acc):
        cand = acc | lax.shift_left(jnp.int32(1), jnp.int32(14) - t)
        cnt = jnp.sum((eq & (nidx < cand)).astype(jnp.int32))
        return jnp.where(cnt < need, cand, acc)

    mm = lax.fori_loop(0, 15, _idx_step, jnp.int32(0))
    sel = (bits > tau) | (eq & (nidx <= mm))
    scores0 = jnp.where(sel, conf, jnp.float32(0.0))

    lane6 = lax.broadcasted_iota(jnp.int32, (_MAXD, 6), 1)
    riota = lax.broadcasted_iota(jnp.int32, (_MAXD, 6), 0)

    def _nms_step(i, carry):
        scores, rows = carry
        mx = jnp.max(scores)
        j = jnp.min(jnp.where(scores == mx, nidx, jnp.int32(1 << 30)))
        pickf = (nidx == j).astype(jnp.float32)

        bx1 = jnp.sum(x1 * pickf)
        by1 = jnp.sum(y1 * pickf)
        bx2 = jnp.sum(x2 * pickf)
        by2 = jnp.sum(y2 * pickf)
        bc = jnp.sum(cid * pickf)

        offj = bc * _MAXWH
        jx1 = bx1 + offj
        jy1 = by1 + offj
        jx2 = bx2 + offj
        jy2 = by2 + offj
        ia1 = (jx2 - jx1) * (jy2 - jy1)
        ix1 = jnp.maximum(jx1, x1o)
        iy1 = jnp.maximum(jy1, y1o)
        ix2 = jnp.minimum(jx2, x2o)
        iy2 = jnp.minimum(jy2, y2o)
        inter = jnp.maximum(ix2 - ix1, 0.0) * jnp.maximum(iy2 - iy1, 0.0)
        iou = inter / (ia1 + a2 - inter + jnp.float32(1e-7))

        valid = jnp.where((mx > 0.0) | (i < 1), jnp.float32(1.0),
                          jnp.float32(0.0))
        rowb = jnp.where(lane6 == 0, bx1,
               jnp.where(lane6 == 1, by1,
               jnp.where(lane6 == 2, bx2,
               jnp.where(lane6 == 3, by2,
               jnp.where(lane6 == 4, mx, bc)))))
        rows = jnp.where(riota == i, rowb * valid, rows)

        scores = jnp.where(iou <= _IOU, scores, jnp.float32(0.0))
        return scores, rows

    _, rows = lax.fori_loop(
        0, _MAXD, _nms_step,
        (scores0, jnp.zeros((_MAXD, 6), jnp.float32)))
    o_ref[...] = rows


def kernel(pred_boxes, fts):
    b, n, c = pred_boxes.shape
    pp = jnp.pad(pred_boxes, ((0, 0), (0, _NPAD - n), (0, 0)))
    pt = pp.reshape(b, _R, _LANES, c).transpose(0, 3, 1, 2)
    dets = pl.pallas_call(
        _nms_body,
        grid=(b,),
        in_specs=[pl.BlockSpec((None, c, _R, _LANES),
                               lambda i: (i, 0, 0, 0))],
        out_specs=pl.BlockSpec((None, _MAXD, 6), lambda i: (i, 0, 0)),
        out_shape=jax.ShapeDtypeStruct((b, _MAXD, 6), jnp.float32),
    )(pt)
    return dets, fts


# speedup vs baseline: 2.5155x; 1.0003x over previous
import jax
import jax.numpy as jnp
from jax import lax
from jax.experimental import pallas as pl
from jax.experimental.pallas import tpu as pltpu

_CONF = 0.4
_IOU = 0.5
_MAXD = 100
_K = 4096
_IMG = 640.0
_MAXWH = 4096.0
_LANES = 128
_R = 200
_NPAD = _R * _LANES


def _nms_body(p_ref, o_ref):
    p = p_ref[...]
    obj = p[4]
    ca = obj[None, :, :] * p[5:85]
    m = jnp.max(ca, axis=0)
    ci = lax.broadcasted_iota(jnp.int32, ca.shape, 0).astype(jnp.float32)
    cid = jnp.min(jnp.where(ca == m[None], ci, jnp.float32(1e9)), axis=0)
    conf = jnp.where(m > _CONF, m, jnp.float32(0.0))

    xs = p[0] * _IMG
    ys = p[1] * _IMG
    ws = p[2] * _IMG
    hs = p[3] * _IMG
    x1 = xs - ws / 2
    y1 = ys - hs / 2
    x2 = xs + ws / 2
    y2 = ys + hs / 2
    off = cid * _MAXWH
    x1o = x1 + off
    y1o = y1 + off
    x2o = x2 + off
    y2o = y2 + off
    a2 = (x2o - x1o) * (y2o - y1o)

    nidx = (lax.broadcasted_iota(jnp.int32, (_R, _LANES), 0) * _LANES
            + lax.broadcasted_iota(jnp.int32, (_R, _LANES), 1))
    bits = lax.bitcast_convert_type(conf, jnp.int32)

    def _tau_step(t, acc):
        cand = acc | lax.shift_left(jnp.int32(1), jnp.int32(30) - t)
        cnt = jnp.sum((bits >= cand).astype(jnp.int32))
        return jnp.where(cnt >= _K, cand, acc)

    tau = lax.fori_loop(0, 31, _tau_step, jnp.int32(0))
    count_gt = jnp.sum((bits > tau).astype(jnp.int32))
    need = _K - count_gt
    eq = bits == tau

    def _idx_step(t, acc):
        cand = acc | lax.shift_left(jnp.int32(1), jnp.int32(14) - t)
        cnt = jnp.sum((eq & (nidx < cand)).astype(jnp.int32))
        return jnp.where(cnt < need, cand, acc)

    mm = lax.fori_loop(0, 15, _idx_step, jnp.int32(0))
    sel = (bits > tau) | (eq & (nidx <= mm))
    scores0 = jnp.where(sel, conf, jnp.float32(0.0))

    lane6 = lax.broadcasted_iota(jnp.int32, (_MAXD, 6), 1)
    riota = lax.broadcasted_iota(jnp.int32, (_MAXD, 6), 0)

    def _nms_step(i, carry):
        scores, rows = carry
        mx = jnp.max(scores)
        j = jnp.min(jnp.where(scores == mx, nidx, jnp.int32(1 << 30)))
        pickf = (nidx == j).astype(jnp.float32)

        bx1 = jnp.sum(x1 * pickf)
        by1 = jnp.sum(y1 * pickf)
        bx2 = jnp.sum(x2 * pickf)
        by2 = jnp.sum(y2 * pickf)
        bc = jnp.sum(cid * pickf)

        offj = bc * _MAXWH
        jx1 = bx1 + offj
        jy1 = by1 + offj
        jx2 = bx2 + offj
        jy2 = by2 + offj
        ia1 = (jx2 - jx1) * (jy2 - jy1)
        ix1 = jnp.maximum(jx1, x1o)
        iy1 = jnp.maximum(jy1, y1o)
        ix2 = jnp.minimum(jx2, x2o)
        iy2 = jnp.minimum(jy2, y2o)
        inter = jnp.maximum(ix2 - ix1, 0.0) * jnp.maximum(iy2 - iy1, 0.0)
        iou = inter / (ia1 + a2 - inter + jnp.float32(1e-7))

        valid = jnp.where((mx > 0.0) | (i < 1), jnp.float32(1.0),
                          jnp.float32(0.0))
        rowb = jnp.where(lane6 == 0, bx1,
               jnp.where(lane6 == 1, by1,
               jnp.where(lane6 == 2, bx2,
               jnp.where(lane6 == 3, by2,
               jnp.where(lane6 == 4, mx, bc)))))
        rows = jnp.where(riota == i, rowb * valid, rows)

        scores = jnp.where(iou <= _IOU, scores, jnp.float32(0.0))
        return scores, rows

    _, rows = lax.fori_loop(
        0, _MAXD, _nms_step,
        (scores0, jnp.zeros((_MAXD, 6), jnp.float32)))
    o_ref[...] = rows


def kernel(pred_boxes, fts):
    b, n, c = pred_boxes.shape
    pp = jnp.pad(pred_boxes, ((0, 0), (0, _NPAD - n), (0, 0)))
    pt = pp.reshape(b, _R, _LANES, c).transpose(0, 3, 1, 2)
    dets = pl.pallas_call(
        _nms_body,
        grid=(b,),
        in_specs=[pl.BlockSpec((None, c, _R, _LANES),
                               lambda i: (i, 0, 0, 0))],
        out_specs=pl.BlockSpec((None, _MAXD, 6), lambda i: (i, 0, 0)),
        out_shape=jax.ShapeDtypeStruct((b, _MAXD, 6), jnp.float32),
        compiler_params=pltpu.CompilerParams(
            dimension_semantics=("parallel",)),
    )(pt)
    return dets, fts


# speedup vs baseline: 2.6537x; 1.0549x over previous
import jax
import jax.numpy as jnp
from jax import lax
from jax.experimental import pallas as pl
from jax.experimental.pallas import tpu as pltpu

_CONF = 0.4
_IOU = 0.5
_MAXD = 100
_K = 4096
_IMG = 640.0
_MAXWH = 4096.0
_LANES = 128
_R = 200
_NPAD = _R * _LANES


def _nms_body(p_ref, o_ref, sx1, sy1, sx2, sy2, scd):
    p = p_ref[...]
    obj = p[4]
    ca = obj[None, :, :] * p[5:85]
    m = jnp.max(ca, axis=0)
    ci = lax.broadcasted_iota(jnp.int32, ca.shape, 0).astype(jnp.float32)
    cid = jnp.min(jnp.where(ca == m[None], ci, jnp.float32(1e9)), axis=0)
    conf = jnp.where(m > _CONF, m, jnp.float32(0.0))

    xs = p[0] * _IMG
    ys = p[1] * _IMG
    ws = p[2] * _IMG
    hs = p[3] * _IMG
    x1 = xs - ws / 2
    y1 = ys - hs / 2
    x2 = xs + ws / 2
    y2 = ys + hs / 2
    off = cid * _MAXWH
    x1o = x1 + off
    y1o = y1 + off
    x2o = x2 + off
    y2o = y2 + off
    a2 = (x2o - x1o) * (y2o - y1o)

    nidx = (lax.broadcasted_iota(jnp.int32, (_R, _LANES), 0) * _LANES
            + lax.broadcasted_iota(jnp.int32, (_R, _LANES), 1))
    bits = lax.bitcast_convert_type(conf, jnp.int32)

    def _tau_step(t, acc):
        cand = acc | lax.shift_left(jnp.int32(1), jnp.int32(30) - t)
        cnt = jnp.sum((bits >= cand).astype(jnp.int32))
        return jnp.where(cnt >= _K, cand, acc)

    tau = lax.fori_loop(0, 31, _tau_step, jnp.int32(0))
    count_gt = jnp.sum((bits > tau).astype(jnp.int32))
    need = _K - count_gt
    eq = bits == tau

    def _idx_step(t, acc):
        cand = acc | lax.shift_left(jnp.int32(1), jnp.int32(14) - t)
        cnt = jnp.sum((eq & (nidx < cand)).astype(jnp.int32))
        return jnp.where(cnt < need, cand, acc)

    mm = lax.fori_loop(0, 15, _idx_step, jnp.int32(0))
    sel = (bits > tau) | (eq & (nidx <= mm))
    scores0 = jnp.where(sel, conf, jnp.float32(0.0))

    sx1[...] = x1
    sy1[...] = y1
    sx2[...] = x2
    sy2[...] = y2
    scd[...] = cid

    lane6 = lax.broadcasted_iota(jnp.int32, (1, 6), 1)
    lane128 = lax.broadcasted_iota(jnp.int32, (1, _LANES), 1)

    def _nms_step(i, scores):
        mx = jnp.max(scores)
        j = jnp.min(jnp.where(scores == mx, nidx, jnp.int32(1 << 30)))
        r = lax.shift_right_logical(j, 7)
        lm = (lane128 == (j & 127)).astype(jnp.float32)

        bx1 = jnp.sum(sx1[pl.ds(r, 1), :] * lm)
        by1 = jnp.sum(sy1[pl.ds(r, 1), :] * lm)
        bx2 = jnp.sum(sx2[pl.ds(r, 1), :] * lm)
        by2 = jnp.sum(sy2[pl.ds(r, 1), :] * lm)
        bc = jnp.sum(scd[pl.ds(r, 1), :] * lm)

        offj = bc * _MAXWH
        jx1 = bx1 + offj
        jy1 = by1 + offj
        jx2 = bx2 + offj
        jy2 = by2 + offj
        ia1 = (jx2 - jx1) * (jy2 - jy1)
        ix1 = jnp.maximum(jx1, x1o)
        iy1 = jnp.maximum(jy1, y1o)
        ix2 = jnp.minimum(jx2, x2o)
        iy2 = jnp.minimum(jy2, y2o)
        inter = jnp.maximum(ix2 - ix1, 0.0) * jnp.maximum(iy2 - iy1, 0.0)
        iou = inter / (ia1 + a2 - inter + jnp.float32(1e-7))

        valid = jnp.where((mx > 0.0) | (i < 1), jnp.float32(1.0),
                          jnp.float32(0.0))
        rowb = jnp.where(lane6 == 0, bx1,
               jnp.where(lane6 == 1, by1,
               jnp.where(lane6 == 2, bx2,
               jnp.where(lane6 == 3, by2,
               jnp.where(lane6 == 4, mx, bc)))))
        o_ref[pl.ds(i, 1), :] = rowb * valid

        return jnp.where(iou <= _IOU, scores, jnp.float32(0.0))

    lax.fori_loop(0, _MAXD, _nms_step, scores0)


def kernel(pred_boxes, fts):
    b, n, c = pred_boxes.shape
    pp = jnp.pad(pred_boxes, ((0, 0), (0, _NPAD - n), (0, 0)))
    pt = pp.reshape(b, _R, _LANES, c).transpose(0, 3, 1, 2)
    dets = pl.pallas_call(
        _nms_body,
        grid=(b,),
        in_specs=[pl.BlockSpec((None, c, _R, _LANES),
                               lambda i: (i, 0, 0, 0))],
        out_specs=pl.BlockSpec((None, _MAXD, 6), lambda i: (i, 0, 0)),
        out_shape=jax.ShapeDtypeStruct((b, _MAXD, 6), jnp.float32),
        scratch_shapes=[pltpu.VMEM((_R, _LANES), jnp.float32)
                        for _ in range(5)],
        compiler_params=pltpu.CompilerParams(
            dimension_semantics=("arbitrary",)),
    )(pt)
    return dets, fts


# speedup vs baseline: 2.9720x; 1.1199x over previous
import jax
import jax.numpy as jnp
from jax import lax
from jax.experimental import pallas as pl
from jax.experimental.pallas import tpu as pltpu

_CONF = 0.4
_IOU = 0.5
_MAXD = 100
_K = 4096
_IMG = 640.0
_MAXWH = 4096.0
_LANES = 128
_R = 200
_NPAD = _R * _LANES
_B = 4


def _nms_body(p_ref, o_ref, sx1, sy1, sx2, sy2, scd):
    nidx = (lax.broadcasted_iota(jnp.int32, (_R, _LANES), 0) * _LANES
            + lax.broadcasted_iota(jnp.int32, (_R, _LANES), 1))
    lane6 = lax.broadcasted_iota(jnp.int32, (1, 6), 1)
    lane128 = lax.broadcasted_iota(jnp.int32, (1, _LANES), 1)

    objs = [p_ref[k, 4] for k in range(_B)]

    def _cls_step(c, carry):
        ms, cids = carry
        cf = c.astype(jnp.float32)
        nm, nc = [], []
        for k in range(_B):
            v = objs[k] * p_ref[k, 5 + c]
            better = v > ms[k]
            nm.append(jnp.where(better, v, ms[k]))
            nc.append(jnp.where(better, cf, cids[k]))
        return tuple(nm), tuple(nc)

    ms0 = tuple(objs[k] * p_ref[k, 5] for k in range(_B))
    cid0 = tuple(jnp.zeros((_R, _LANES), jnp.float32) for _ in range(_B))
    ms, cids = lax.fori_loop(1, 80, _cls_step, (ms0, cid0))

    confs = [jnp.where(ms[k] > _CONF, ms[k], jnp.float32(0.0))
             for k in range(_B)]
    bitss = [lax.bitcast_convert_type(confs[k], jnp.int32) for k in range(_B)]

    def _tau_step(t, taus):
        out = []
        for k in range(_B):
            cand = taus[k] | lax.shift_left(jnp.int32(1), jnp.int32(30) - t)
            cnt = jnp.sum((bitss[k] >= cand).astype(jnp.int32))
            out.append(jnp.where(cnt >= _K, cand, taus[k]))
        return tuple(out)

    taus = lax.fori_loop(0, 31, _tau_step,
                         tuple(jnp.int32(0) for _ in range(_B)))
    eqs = [bitss[k] == taus[k] for k in range(_B)]
    needs = [_K - jnp.sum((bitss[k] > taus[k]).astype(jnp.int32))
             for k in range(_B)]

    def _idx_step(t, mms):
        out = []
        for k in range(_B):
            cand = mms[k] | lax.shift_left(jnp.int32(1), jnp.int32(14) - t)
            cnt = jnp.sum((eqs[k] & (nidx < cand)).astype(jnp.int32))
            out.append(jnp.where(cnt < needs[k], cand, mms[k]))
        return tuple(out)

    mms = lax.fori_loop(0, 15, _idx_step,
                        tuple(jnp.int32(0) for _ in range(_B)))

    scores0 = []
    consts = []
    for k in range(_B):
        xs = p_ref[k, 0] * _IMG
        ys = p_ref[k, 1] * _IMG
        ws = p_ref[k, 2] * _IMG
        hs = p_ref[k, 3] * _IMG
        x1 = xs - ws / 2
        y1 = ys - hs / 2
        x2 = xs + ws / 2
        y2 = ys + hs / 2
        off = cids[k] * _MAXWH
        x1o = x1 + off
        y1o = y1 + off
        x2o = x2 + off
        y2o = y2 + off
        a2 = (x2o - x1o) * (y2o - y1o)
        sx1[k] = x1
        sy1[k] = y1
        sx2[k] = x2
        sy2[k] = y2
        scd[k] = cids[k]
        sel = (bitss[k] > taus[k]) | (eqs[k] & (nidx <= mms[k]))
        scores0.append(jnp.where(sel, confs[k], jnp.float32(0.0)))
        consts.append((x1o, y1o, x2o, y2o, a2))

    def _one(i, k, scores):
        x1o, y1o, x2o, y2o, a2 = consts[k]
        mx = jnp.max(scores)
        j = jnp.min(jnp.where(scores == mx, nidx, jnp.int32(1 << 30)))
        r = lax.shift_right_logical(j, 7)
        lm = (lane128 == (j & 127)).astype(jnp.float32)

        bx1 = jnp.sum(sx1[k, pl.ds(r, 1), :] * lm)
        by1 = jnp.sum(sy1[k, pl.ds(r, 1), :] * lm)
        bx2 = jnp.sum(sx2[k, pl.ds(r, 1), :] * lm)
        by2 = jnp.sum(sy2[k, pl.ds(r, 1), :] * lm)
        bc = jnp.sum(scd[k, pl.ds(r, 1), :] * lm)

        offj = bc * _MAXWH
        jx1 = bx1 + offj
        jy1 = by1 + offj
        jx2 = bx2 + offj
        jy2 = by2 + offj
        ia1 = (jx2 - jx1) * (jy2 - jy1)
        ix1 = jnp.maximum(jx1, x1o)
        iy1 = jnp.maximum(jy1, y1o)
        ix2 = jnp.minimum(jx2, x2o)
        iy2 = jnp.minimum(jy2, y2o)
        inter = jnp.maximum(ix2 - ix1, 0.0) * jnp.maximum(iy2 - iy1, 0.0)
        iou = inter / (ia1 + a2 - inter + jnp.float32(1e-7))

        valid = jnp.where((mx > 0.0) | (i < 1), jnp.float32(1.0),
                          jnp.float32(0.0))
        rowb = jnp.where(lane6 == 0, bx1,
               jnp.where(lane6 == 1, by1,
               jnp.where(lane6 == 2, bx2,
               jnp.where(lane6 == 3, by2,
               jnp.where(lane6 == 4, mx, bc)))))
        o_ref[k, pl.ds(i, 1), :] = rowb * valid

        return jnp.where(iou <= _IOU, scores, jnp.float32(0.0))

    def _nms_step(i, carry):
        return tuple(_one(i, k, carry[k]) for k in range(_B))

    lax.fori_loop(0, _MAXD, _nms_step, tuple(scores0))


def kernel(pred_boxes, fts):
    b, n, c = pred_boxes.shape
    pp = jnp.pad(pred_boxes, ((0, 0), (0, _NPAD - n), (0, 0)))
    pt = pp.reshape(b, _R, _LANES, c).transpose(0, 3, 1, 2)
    dets = pl.pallas_call(
        _nms_body,
        grid=(1,),
        in_specs=[pl.BlockSpec((_B, c, _R, _LANES),
                               lambda i: (0, 0, 0, 0))],
        out_specs=pl.BlockSpec((_B, _MAXD, 6), lambda i: (0, 0, 0)),
        out_shape=jax.ShapeDtypeStruct((b, _MAXD, 6), jnp.float32),
        scratch_shapes=[pltpu.VMEM((_B, _R, _LANES), jnp.float32)
                        for _ in range(5)],
        compiler_params=pltpu.CompilerParams(
            dimension_semantics=("arbitrary",)),
    )(pt)
    return dets, fts


# speedup vs baseline: 3.2305x; 1.0870x over previous
import jax
import jax.numpy as jnp
from jax import lax
from jax.experimental import pallas as pl
from jax.experimental.pallas import tpu as pltpu

_CONF = 0.4
_IOU = 0.5
_MAXD = 100
_K = 4096
_IMG = 640.0
_MAXWH = 4096.0
_LANES = 128
_R = 200
_NPAD = _R * _LANES
_B = 4


def _nms_body(p_ref, o_ref, sx1, sy1, sx2, sy2, scd):
    nidx = (lax.broadcasted_iota(jnp.int32, (_R, _LANES), 0) * _LANES
            + lax.broadcasted_iota(jnp.int32, (_R, _LANES), 1))
    lane6 = lax.broadcasted_iota(jnp.int32, (1, 6), 1)
    lane128 = lax.broadcasted_iota(jnp.int32, (1, _LANES), 1)

    objs = [p_ref[k, 4] for k in range(_B)]

    def _cls_step(c, carry):
        ms, cids = carry
        cf = c.astype(jnp.float32)
        nm, nc = [], []
        for k in range(_B):
            v = objs[k] * p_ref[k, 5 + c]
            better = v > ms[k]
            nm.append(jnp.where(better, v, ms[k]))
            nc.append(jnp.where(better, cf, cids[k]))
        return tuple(nm), tuple(nc)

    ms0 = tuple(objs[k] * p_ref[k, 5] for k in range(_B))
    cid0 = tuple(jnp.zeros((_R, _LANES), jnp.float32) for _ in range(_B))
    ms, cids = lax.fori_loop(1, 80, _cls_step, (ms0, cid0))

    confs = [jnp.where(ms[k] > _CONF, ms[k], jnp.float32(0.0))
             for k in range(_B)]
    bitss = [lax.bitcast_convert_type(confs[k], jnp.int32) for k in range(_B)]

    def _tau_step(t, taus):
        out = []
        for k in range(_B):
            cand = taus[k] | lax.shift_left(jnp.int32(1), jnp.int32(30) - t)
            cnt = jnp.sum((bitss[k] >= cand).astype(jnp.int32))
            out.append(jnp.where(cnt >= _K, cand, taus[k]))
        return tuple(out)

    taus = lax.fori_loop(0, 31, _tau_step,
                         tuple(jnp.int32(0) for _ in range(_B)))
    eqs = [bitss[k] == taus[k] for k in range(_B)]
    needs = [_K - jnp.sum((bitss[k] > taus[k]).astype(jnp.int32))
             for k in range(_B)]

    def _idx_step(t, mms):
        out = []
        for k in range(_B):
            cand = mms[k] | lax.shift_left(jnp.int32(1), jnp.int32(14) - t)
            cnt = jnp.sum((eqs[k] & (nidx < cand)).astype(jnp.int32))
            out.append(jnp.where(cnt < needs[k], cand, mms[k]))
        return tuple(out)

    mms = lax.fori_loop(0, 15, _idx_step,
                        tuple(jnp.int32(0) for _ in range(_B)))

    scores0 = []
    consts = []
    for k in range(_B):
        xs = p_ref[k, 0] * _IMG
        ys = p_ref[k, 1] * _IMG
        ws = p_ref[k, 2] * _IMG
        hs = p_ref[k, 3] * _IMG
        x1 = xs - ws / 2
        y1 = ys - hs / 2
        x2 = xs + ws / 2
        y2 = ys + hs / 2
        off = cids[k] * _MAXWH
        x1o = x1 + off
        y1o = y1 + off
        x2o = x2 + off
        y2o = y2 + off
        a2 = (x2o - x1o) * (y2o - y1o)
        sx1[k] = x1
        sy1[k] = y1
        sx2[k] = x2
        sy2[k] = y2
        scd[k] = cids[k]
        sel = (bitss[k] > taus[k]) | (eqs[k] & (nidx <= mms[k]))
        scores0.append(jnp.where(sel, confs[k], jnp.float32(0.0)))
        consts.append((x1o, y1o, x2o, y2o, a2))

    def _one(i, k, scores):
        x1o, y1o, x2o, y2o, a2 = consts[k]
        mx = jnp.max(scores)
        j = jnp.min(jnp.where(scores == mx, nidx, jnp.int32(1 << 30)))
        r = lax.shift_right_logical(j, 7)
        lm = (lane128 == (j & 127)).astype(jnp.float32)

        bx1 = jnp.sum(sx1[k, pl.ds(r, 1), :] * lm)
        by1 = jnp.sum(sy1[k, pl.ds(r, 1), :] * lm)
        bx2 = jnp.sum(sx2[k, pl.ds(r, 1), :] * lm)
        by2 = jnp.sum(sy2[k, pl.ds(r, 1), :] * lm)
        bc = jnp.sum(scd[k, pl.ds(r, 1), :] * lm)

        offj = bc * _MAXWH
        jx1 = bx1 + offj
        jy1 = by1 + offj
        jx2 = bx2 + offj
        jy2 = by2 + offj
        ia1 = (jx2 - jx1) * (jy2 - jy1)
        ix1 = jnp.maximum(jx1, x1o)
        iy1 = jnp.maximum(jy1, y1o)
        ix2 = jnp.minimum(jx2, x2o)
        iy2 = jnp.minimum(jy2, y2o)
        inter = jnp.maximum(ix2 - ix1, 0.0) * jnp.maximum(iy2 - iy1, 0.0)
        iou = inter / (ia1 + a2 - inter + jnp.float32(1e-7))

        valid = jnp.where((mx > 0.0) | (i < 1), jnp.float32(1.0),
                          jnp.float32(0.0))
        rowb = jnp.where(lane6 == 0, bx1,
               jnp.where(lane6 == 1, by1,
               jnp.where(lane6 == 2, bx2,
               jnp.where(lane6 == 3, by2,
               jnp.where(lane6 == 4, mx, bc)))))
        o_ref[k, pl.ds(i, 1), :] = rowb * valid

        return jnp.where(iou <= _IOU, scores, jnp.float32(0.0))

    def _nms_step(i, carry):
        return tuple(_one(i, k, carry[k]) for k in range(_B))

    lax.fori_loop(0, _MAXD, _nms_step, tuple(scores0))


def kernel(pred_boxes, fts):
    b, n, c = pred_boxes.shape
    pp = jnp.pad(pred_boxes.transpose(0, 2, 1),
                 ((0, 0), (0, 0), (0, _NPAD - n)))
    pt = pp.reshape(b, c, _R, _LANES)
    dets = pl.pallas_call(
        _nms_body,
        grid=(1,),
        in_specs=[pl.BlockSpec((_B, c, _R, _LANES),
                               lambda i: (0, 0, 0, 0))],
        out_specs=pl.BlockSpec((_B, _MAXD, 6), lambda i: (0, 0, 0)),
        out_shape=jax.ShapeDtypeStruct((b, _MAXD, 6), jnp.float32),
        scratch_shapes=[pltpu.VMEM((_B, _R, _LANES), jnp.float32)
                        for _ in range(5)],
        compiler_params=pltpu.CompilerParams(
            dimension_semantics=("arbitrary",)),
    )(pt)
    return dets, fts
